# Initial kernel scaffold; baseline (speedup 1.0000x reference)
import jax
import jax.numpy as jnp
from jax.experimental import pallas as pl


def kernel(x, edges, rels, r, kernels, bn_gamma, bn_beta):
    raise NotImplementedError("write your pallas kernel here")


# speedup vs baseline: 5.1968x; 5.1968x over previous
import functools

import jax
import jax.numpy as jnp
from jax import lax
from jax.experimental import pallas as pl
from jax.experimental.pallas import tpu as pltpu
from jax.experimental.pallas import tpu_sc as plsc

_C = 128
_NW = 32
_EPS = 1e-5


def _sc_message_pass(x, src, dst, hidx, n_cols, per_tile):
    n, d = x.shape
    groups = per_tile // _C
    rows_pad = n + 8
    fl_rows = (n // 16) // 8 * 8
    tail0 = fl_rows * 16
    hflat = rows_pad * n_cols

    mesh = plsc.VectorSubcoreMesh(core_axis_name="c", subcore_axis_name="s")

    @functools.partial(
        pl.kernel,
        out_type=(
            jax.ShapeDtypeStruct((2, n, d), jnp.float32),
            jax.ShapeDtypeStruct((2, n * n_cols), jnp.float32),
        ),
        mesh=mesh,
        scratch_types=(
            pltpu.VMEM((_C,), jnp.int32),
            pltpu.VMEM((_C,), jnp.int32),
            pltpu.VMEM((_C,), jnp.int32),
            pltpu.VMEM((_C, d), jnp.float32),
            pltpu.VMEM((_C,), jnp.float32),
            pltpu.VMEM_SHARED((rows_pad, d), jnp.float32),
            pltpu.VMEM_SHARED((hflat,), jnp.float32),
            pltpu.SemaphoreType.DMA,
        ),
    )
    def run(x_hbm, src_hbm, dst_hbm, hidx_hbm, zr_hbm, zh_hbm, ones_hbm,
            p_hbm, h_hbm,
            src_v, dst_v, hidx_v, rows_v, ones_v, acc_sh, hist_sh, sem):
        c = lax.axis_index("c")
        s = lax.axis_index("s")
        w = c * 16 + s

        pltpu.sync_copy(zr_hbm, rows_v)
        pltpu.sync_copy(ones_hbm, ones_v)
        zb = s * fl_rows
        nchunks = (fl_rows + _C - 1) // _C
        for k in range(nchunks):
            m = min(_C, fl_rows - k * _C)
            pltpu.sync_copy(rows_v.at[pl.ds(0, m)],
                            acc_sh.at[pl.ds(zb + k * _C, m)])

        @pl.when(s == 15)
        def _zero_tail():
            pltpu.sync_copy(rows_v.at[pl.ds(0, rows_pad - tail0)],
                            acc_sh.at[pl.ds(tail0, rows_pad - tail0)])

        @pl.when(s == 0)
        def _zero_hist():
            pltpu.sync_copy(zh_hbm, hist_sh)

        plsc.subcore_barrier()

        base = w * per_tile

        def body(g, carry):
            off = base + g * _C
            pltpu.sync_copy(src_hbm.at[pl.ds(off, _C)], src_v)
            pltpu.sync_copy(dst_hbm.at[pl.ds(off, _C)], dst_v)
            pltpu.sync_copy(hidx_hbm.at[pl.ds(off, _C)], hidx_v)
            pltpu.async_copy(x_hbm.at[src_v], rows_v, sem).wait()
            pltpu.sync_copy(rows_v, acc_sh.at[dst_v], add=True)
            pltpu.sync_copy(ones_v, hist_sh.at[hidx_v], add=True)
            return carry

        lax.fori_loop(0, groups, body, 0)
        plsc.subcore_barrier()

        fb = s * fl_rows
        pltpu.sync_copy(acc_sh.at[pl.ds(fb, fl_rows)],
                        p_hbm.at[c, pl.ds(fb, fl_rows)])
        pltpu.sync_copy(hist_sh.at[pl.ds(fb * n_cols, fl_rows * n_cols)],
                        h_hbm.at[c, pl.ds(fb * n_cols, fl_rows * n_cols)])

        @pl.when(s == 15)
        def _flush_tail():
            pltpu.sync_copy(acc_sh.at[pl.ds(tail0, n - tail0)],
                            p_hbm.at[c, pl.ds(tail0, n - tail0)])
            pltpu.sync_copy(
                hist_sh.at[pl.ds(tail0 * n_cols, (n - tail0) * n_cols)],
                h_hbm.at[c, pl.ds(tail0 * n_cols, (n - tail0) * n_cols)])

    zr = jnp.zeros((_C, d), jnp.float32)
    zh = jnp.zeros((hflat,), jnp.float32)
    on = jnp.ones((_C,), jnp.float32)
    return run(x, src, dst, hidx, zr, zh, on)


def _tc_combine(x, p, h, r, kernels, bn_gamma, bn_beta, escal):
    n, d = x.shape

    def body(x_ref, p_ref, h_ref, r_ref, k_ref, g_ref, b_ref, es_ref, o_ref):
        xv = x_ref[...]
        mean = jnp.mean(xv, axis=0)
        xc = xv - mean[None, :]
        var = jnp.mean(xc * xc, axis=0)
        kv = k_ref[0, :]
        sc = g_ref[0, :] / jnp.sqrt(var + _EPS)
        a = sc * kv
        b = (b_ref[0, :] - mean * sc) * kv
        rk = r_ref[...] * kv[None, :]
        t = jnp.concatenate([rk, -rk], axis=0) + b[None, :]
        pv = p_ref[0] + p_ref[1]
        hv = h_ref[0] + h_ref[1]
        relpart = jnp.dot(hv, t, preferred_element_type=jnp.float32,
                          precision=lax.Precision.HIGHEST)
        xk = xv * a[None, :] + b[None, :]
        num = pv * a[None, :] + relpart + xk
        esv = es_ref[...]
        iot = lax.broadcasted_iota(jnp.int32, (n, 1), 0)
        du = jnp.ones((n, 1), jnp.float32)
        for i in range(3):
            ai = esv[2 * i:2 * i + 1, :]
            bi = esv[2 * i + 1:2 * i + 2, :]
            inc = (ai != bi).astype(jnp.float32)
            du = du + inc * ((iot == ai).astype(jnp.float32)
                             + (iot == bi).astype(jnp.float32))
        o_ref[...] = num / du

    return pl.pallas_call(
        body,
        out_shape=jax.ShapeDtypeStruct((n, d), jnp.float32),
    )(x, p, h, r, kernels, bn_gamma, bn_beta, escal)


def kernel(x, edges, rels, r, kernels, bn_gamma, bn_beta):
    n, d = x.shape
    e = edges.shape[1]
    nrel = r.shape[0]
    e0 = edges[0].astype(jnp.int32)
    e1 = edges[1].astype(jnp.int32)
    rl = rels.astype(jnp.int32)
    m = 2 * e
    per_tile = -(-m // (_NW * _C)) * _C
    pad = per_tile * _NW - m
    n_cols = 2 * nrel
    src = jnp.concatenate([e0, e1, jnp.zeros((pad,), jnp.int32)])
    dst = jnp.concatenate([e1, e0, jnp.full((pad,), n, jnp.int32)])
    col = jnp.concatenate([rl, rl + nrel, jnp.zeros((pad,), jnp.int32)])
    hidx = dst * n_cols + col
    p, hf = _sc_message_pass(x, src, dst, hidx, n_cols, per_tile)
    h = hf.reshape(2, n, n_cols)
    escal = jnp.stack([e0[0], e0[2], rl[0], rl[2], e1[0], e1[2],
                       jnp.zeros((), jnp.int32), jnp.zeros((), jnp.int32)])
    return _tc_combine(x, p, h, r, kernels,
                       bn_gamma.reshape(1, d), bn_beta.reshape(1, d),
                       escal.reshape(8, 1))
